# Initial kernel scaffold; baseline (speedup 1.0000x reference)
import jax
import jax.numpy as jnp
from jax.experimental import pallas as pl


def kernel(nodes, edges, senders, receivers, global_latent, node_graph_idx, W, b):
    raise NotImplementedError("write your pallas kernel here")


# speedup vs baseline: 2.4191x; 2.4191x over previous
import functools

import jax
import jax.numpy as jnp
from jax import lax
from jax.experimental import pallas as pl
from jax.experimental.pallas import tpu as pltpu
from jax.experimental.pallas import tpu_sc as plsc

NC = 2
NS = 16
L = 16
NW = NC * NS
D = 256
G = 16
DC = D // L
N = 50000
CHUNK = 1560
NBLK = 13
BR = CHUNK // NBLK
TAIL = N - NW * CHUNK

_mesh = plsc.VectorSubcoreMesh(core_axis_name="c", subcore_axis_name="s")


@functools.partial(
    pl.kernel,
    out_type=jax.ShapeDtypeStruct((NC, G, D), jnp.float32),
    mesh=_mesh,
    scratch_types=[
        pltpu.VMEM((CHUNK + L,), jnp.int32),
        pltpu.VMEM((TAIL + L,), jnp.int32),
        pltpu.VMEM((BR, D), jnp.float32),
        pltpu.VMEM((G, D), jnp.float32),
        pltpu.VMEM((G, D), jnp.float32),
        pltpu.VMEM_SHARED((NS, G, D), jnp.float32),
    ],
)
def _segment_pool(nodes_h, idx_h, out_h, idx_v, idx_t, buf_v, acc_v, tmp_v,
                  shared):
    cid = lax.axis_index("c")
    sid = lax.axis_index("s")
    wid = cid * NS + sid
    base = wid * CHUNK

    zeros = jnp.zeros((L,), jnp.float32)

    def _zero(i, _):
        for c in range(DC):
            acc_v[i, pl.ds(c * L, L)] = zeros
        return 0

    lax.fori_loop(0, G, _zero, 0)

    pltpu.sync_copy(idx_h.at[pl.ds(base, CHUNK)], idx_v.at[pl.ds(0, CHUNK)])

    def _accum_rows(idx_ref, idx_off, n_rows):
        def _row(i, _):
            g = idx_ref[pl.ds(idx_off + i, L)][0]
            for c in range(DC):
                plsc.addupdate(acc_v.at[g, pl.ds(c * L, L)],
                               buf_v[i, pl.ds(c * L, L)])
            return 0

        lax.fori_loop(0, n_rows, _row, 0)

    for blk in range(NBLK):
        pltpu.sync_copy(nodes_h.at[pl.ds(base + blk * BR, BR)], buf_v)
        _accum_rows(idx_v, blk * BR, BR)

    @pl.when(wid == NW - 1)
    def _():
        pltpu.sync_copy(idx_h.at[pl.ds(NW * CHUNK, TAIL)],
                        idx_t.at[pl.ds(0, TAIL)])
        pltpu.sync_copy(nodes_h.at[pl.ds(NW * CHUNK, TAIL)],
                        buf_v.at[pl.ds(0, TAIL)])
        _accum_rows(idx_t, 0, TAIL)

    pltpu.sync_copy(acc_v, shared.at[sid])
    plsc.subcore_barrier()

    def _acc_add(i, _):
        for c in range(DC):
            acc_v[i, pl.ds(c * L, L)] = (acc_v[i, pl.ds(c * L, L)] +
                                         tmp_v[i, pl.ds(c * L, L)])
        return 0

    for step in (8, 4, 2, 1):
        @pl.when(sid < step)
        def _(step=step):
            pltpu.sync_copy(shared.at[sid + step], tmp_v)
            lax.fori_loop(0, G, _acc_add, 0)
            pltpu.sync_copy(acc_v, shared.at[sid])

        plsc.subcore_barrier()

    @pl.when(sid == 0)
    def _():
        pltpu.sync_copy(acc_v, out_h.at[cid])


@functools.partial(
    pl.kernel,
    out_type=jax.ShapeDtypeStruct((G,), jnp.float32),
    mesh=_mesh,
    scratch_types=[
        pltpu.VMEM((NC, G, D), jnp.float32),
        pltpu.VMEM((G, D), jnp.float32),
        pltpu.VMEM((D,), jnp.float32),
        pltpu.VMEM((D,), jnp.float32),
        pltpu.VMEM((G,), jnp.float32),
        pltpu.VMEM((G,), jnp.float32),
    ],
)
def _decode(parts_h, glob_h, wp_h, wg_h, b_h, out_h, parts_v, glob_v, wp_v,
            wg_v, b_v, out_v):
    cid = lax.axis_index("c")
    sid = lax.axis_index("s")

    @pl.when((cid == 0) & (sid == 0))
    def _():
        pltpu.sync_copy(parts_h, parts_v)
        pltpu.sync_copy(glob_h, glob_v)
        pltpu.sync_copy(wp_h, wp_v)
        pltpu.sync_copy(wg_h, wg_v)
        pltpu.sync_copy(b_h, b_v)

        lane = lax.iota(jnp.int32, L)
        lv = b_v[...]
        for g in range(G):
            def _c(c, pv, g=g):
                p = (parts_v[0, g, pl.ds(c * L, L)] +
                     parts_v[1, g, pl.ds(c * L, L)])
                pv = pv + p * wp_v[pl.ds(c * L, L)]
                pv = pv + (glob_v[g, pl.ds(c * L, L)] *
                           wg_v[pl.ds(c * L, L)])
                return pv

            pv = lax.fori_loop(0, DC, _c, jnp.zeros((L,), jnp.float32))
            s = pv[0]
            for j in range(1, L):
                s = s + pv[j]
            lv = jnp.where(lane == g, lv + s, lv)
        out_v[...] = lv
        pltpu.sync_copy(out_v, out_h)


def kernel(nodes, edges, senders, receivers, global_latent, node_graph_idx,
           W, b):
    idx = node_graph_idx.astype(jnp.int32)
    parts = _segment_pool(nodes, idx)
    wp = W[:D, 0].astype(jnp.float32)
    wg = W[D:, 0].astype(jnp.float32)
    bb = jnp.broadcast_to(b.astype(jnp.float32), (G,))
    logits = _decode(parts, global_latent, wp, wg, bb)
    return logits.reshape(G, 1)


# speedup vs baseline: 2.7731x; 1.1463x over previous
import functools

import jax
import jax.numpy as jnp
from jax import lax
from jax.experimental import pallas as pl
from jax.experimental.pallas import tpu as pltpu
from jax.experimental.pallas import tpu_sc as plsc

NC = 2
NS = 16
L = 16
NW = NC * NS
D = 256
G = 16
DC = D // L
N = 50000
CHUNK = 1560
NBLK = 13
BR = CHUNK // NBLK
TAIL = N - NW * CHUNK

_mesh = plsc.VectorSubcoreMesh(core_axis_name="c", subcore_axis_name="s")


@functools.partial(
    pl.kernel,
    out_type=jax.ShapeDtypeStruct((NC, G, D), jnp.float32),
    mesh=_mesh,
    scratch_types=[
        pltpu.VMEM((CHUNK + L,), jnp.int32),
        pltpu.VMEM((TAIL + L,), jnp.int32),
        pltpu.VMEM((2, BR, D), jnp.float32),
        pltpu.VMEM((G, D), jnp.float32),
        pltpu.VMEM((G, D), jnp.float32),
        pltpu.VMEM_SHARED((NS, G, D), jnp.float32),
        pltpu.SemaphoreType.DMA,
        pltpu.SemaphoreType.DMA,
    ],
)
def _segment_pool(nodes_h, idx_h, out_h, idx_v, idx_t, buf_v, acc_v, tmp_v,
                  shared, sem0, sem1):
    cid = lax.axis_index("c")
    sid = lax.axis_index("s")
    wid = cid * NS + sid
    base = wid * CHUNK

    zeros = jnp.zeros((L,), jnp.float32)

    def _zero(i, _):
        for c in range(DC):
            acc_v[i, pl.ds(c * L, L)] = zeros
        return 0

    lax.fori_loop(0, G, _zero, 0)

    pltpu.sync_copy(idx_h.at[pl.ds(base, CHUNK)], idx_v.at[pl.ds(0, CHUNK)])

    def _accum_rows(idx_ref, idx_off, n_rows, b):
        def _row(i, _):
            g = idx_ref[pl.ds(idx_off + i, L)][0]
            for c in range(DC):
                plsc.addupdate(acc_v.at[g, pl.ds(c * L, L)],
                               buf_v[b, i, pl.ds(c * L, L)])
            return 0

        lax.fori_loop(0, n_rows, _row, 0)

    def _node_copy(blk, b):
        return pltpu.make_async_copy(
            nodes_h.at[pl.ds(base + blk * BR, BR)],
            buf_v.at[b],
            sem0 if b == 0 else sem1)

    _node_copy(0, 0).start()
    for blk in range(NBLK):
        b = blk % 2
        cp = _node_copy(blk, b)
        if blk + 1 < NBLK:
            _node_copy(blk + 1, (blk + 1) % 2).start()
        cp.wait()
        _accum_rows(idx_v, blk * BR, BR, b)

    @pl.when(wid == NW - 1)
    def _():
        pltpu.sync_copy(idx_h.at[pl.ds(NW * CHUNK, TAIL)],
                        idx_t.at[pl.ds(0, TAIL)])
        pltpu.sync_copy(nodes_h.at[pl.ds(NW * CHUNK, TAIL)],
                        buf_v.at[0, pl.ds(0, TAIL)])
        _accum_rows(idx_t, 0, TAIL, 0)

    pltpu.sync_copy(acc_v, shared.at[sid])
    plsc.subcore_barrier()

    def _acc_add(i, _):
        for c in range(DC):
            acc_v[i, pl.ds(c * L, L)] = (acc_v[i, pl.ds(c * L, L)] +
                                         tmp_v[i, pl.ds(c * L, L)])
        return 0

    for step in (8, 4, 2, 1):
        @pl.when(sid < step)
        def _(step=step):
            pltpu.sync_copy(shared.at[sid + step], tmp_v)
            lax.fori_loop(0, G, _acc_add, 0)
            pltpu.sync_copy(acc_v, shared.at[sid])

        plsc.subcore_barrier()

    @pl.when(sid == 0)
    def _():
        pltpu.sync_copy(acc_v, out_h.at[cid])


@functools.partial(
    pl.kernel,
    out_type=jax.ShapeDtypeStruct((G,), jnp.float32),
    mesh=_mesh,
    scratch_types=[
        pltpu.VMEM((NC, G, D), jnp.float32),
        pltpu.VMEM((G, D), jnp.float32),
        pltpu.VMEM((D,), jnp.float32),
        pltpu.VMEM((D,), jnp.float32),
        pltpu.VMEM((G,), jnp.float32),
        pltpu.VMEM((G,), jnp.float32),
    ],
)
def _decode(parts_h, glob_h, wp_h, wg_h, b_h, out_h, parts_v, glob_v, wp_v,
            wg_v, b_v, out_v):
    cid = lax.axis_index("c")
    sid = lax.axis_index("s")

    @pl.when((cid == 0) & (sid == 0))
    def _():
        pltpu.sync_copy(parts_h, parts_v)
        pltpu.sync_copy(glob_h, glob_v)
        pltpu.sync_copy(wp_h, wp_v)
        pltpu.sync_copy(wg_h, wg_v)
        pltpu.sync_copy(b_h, b_v)

        lane = lax.iota(jnp.int32, L)
        lv = b_v[...]
        for g in range(G):
            def _c(c, pv, g=g):
                p = (parts_v[0, g, pl.ds(c * L, L)] +
                     parts_v[1, g, pl.ds(c * L, L)])
                pv = pv + p * wp_v[pl.ds(c * L, L)]
                pv = pv + (glob_v[g, pl.ds(c * L, L)] *
                           wg_v[pl.ds(c * L, L)])
                return pv

            pv = lax.fori_loop(0, DC, _c, jnp.zeros((L,), jnp.float32))
            s = pv[0]
            for j in range(1, L):
                s = s + pv[j]
            lv = jnp.where(lane == g, lv + s, lv)
        out_v[...] = lv
        pltpu.sync_copy(out_v, out_h)


def kernel(nodes, edges, senders, receivers, global_latent, node_graph_idx,
           W, b):
    idx = node_graph_idx.astype(jnp.int32)
    parts = _segment_pool(nodes, idx)
    wp = W[:D, 0].astype(jnp.float32)
    wg = W[D:, 0].astype(jnp.float32)
    bb = jnp.broadcast_to(b.astype(jnp.float32), (G,))
    logits = _decode(parts, global_latent, wp, wg, bb)
    return logits.reshape(G, 1)
